# baseline (speedup 1.0000x reference)
import functools

import jax
import jax.numpy as jnp
from jax import lax
from jax.experimental import pallas as pl
from jax.experimental.pallas import tpu as pltpu
from jax.experimental.pallas import tpu_sc as plsc

_info = plsc.get_sparse_core_info()
_NC, _NS = _info.num_cores, _info.num_subcores
_NW = _NC * _NS
_NBUF = 6


def _make_gather(V, D, B, chunk):
    n_chunks = (B // _NW) // chunk
    b_per_w = B // _NW
    mesh = plsc.VectorSubcoreMesh(core_axis_name="c", subcore_axis_name="s")

    @functools.partial(
        pl.kernel,
        mesh=mesh,
        out_type=jax.ShapeDtypeStruct((B, D), jnp.float32),
        scratch_types=[
            pltpu.VMEM((n_chunks, chunk), jnp.int32),
        ]
        + [pltpu.VMEM((chunk, D), jnp.float32)] * _NBUF
        + [pltpu.SemaphoreType.DMA] * (2 * _NBUF),
    )
    def gather(idx_hbm, table_hbm, out_hbm, idx_v, *rest):
        bufs = rest[:_NBUF]
        gsems = rest[_NBUF : 2 * _NBUF]
        ssems = rest[2 * _NBUF : 3 * _NBUF]
        wid = lax.axis_index("s") * _NC + lax.axis_index("c")
        base = wid * b_per_w
        pltpu.sync_copy(idx_hbm.at[wid], idx_v)

        def start_gather(c):
            return pltpu.async_copy(
                table_hbm.at[idx_v.at[c]], bufs[c % _NBUF], gsems[c % _NBUF]
            )

        def start_store(c):
            return pltpu.async_copy(
                bufs[c % _NBUF],
                out_hbm.at[pl.ds(base + c * chunk, chunk)],
                ssems[c % _NBUF],
            )

        g = [None] * n_chunks
        s = [None] * n_chunks
        for c in range(min(_NBUF - 1, n_chunks)):
            g[c] = start_gather(c)
        for c in range(n_chunks):
            g[c].wait()
            s[c] = start_store(c)
            nxt = c + _NBUF - 1
            if nxt < n_chunks:
                if c >= 1:
                    s[c - 1].wait()
                g[nxt] = start_gather(nxt)
        for c in range(max(0, n_chunks - _NBUF), n_chunks):
            if s[c] is not None:
                s[c].wait()

    return gather


def kernel(indices, embedding):
    Bb, T = indices.shape
    V, D = embedding.shape
    B = Bb * T
    chunk = 8
    idx3 = indices.reshape(_NW, (B // _NW) // chunk, chunk).astype(jnp.int32)
    out = _make_gather(V, D, B, chunk)(idx3, embedding)
    return out.reshape(Bb, T, D)
